# baseline (speedup 1.0000x reference)
import functools

import numpy as np
import jax
import jax.numpy as jnp
from jax import lax
from jax.experimental import pallas as pl
from jax.experimental.pallas import tpu as pltpu
from jax.experimental.pallas import tpu_sc as plsc

_N_BINS = 15
_NC = 2
_NS = 16
_L = 16
_NW = _NC * _NS


@functools.cache
def _build_sc_hist(n):
    assert n % _L == 0
    big = ((n + _NW * _L - 1) // (_NW * _L)) * _L
    small = n - (_NW - 1) * big
    assert 0 < small <= big and small % _L == 0
    assert big * 65537 < 2**31
    n_common = small // _L
    n_extra = (big - small) // _L
    unroll = 18
    assert n_common % unroll == 0
    half1 = small
    mesh = plsc.VectorSubcoreMesh(core_axis_name="c", subcore_axis_name="s")

    @functools.partial(
        pl.kernel,
        mesh=mesh,
        compiler_params=pltpu.CompilerParams(needs_layout_passes=False),
        out_type=jax.ShapeDtypeStruct((_NW, 3 * _L), jnp.float32),
        scratch_types=[
            pltpu.VMEM((big,), jnp.float32),
            pltpu.VMEM((big,), jnp.float32),
            pltpu.VMEM((_L,), jnp.float32),
            pltpu.VMEM((_L * 17,), jnp.float32),
            pltpu.VMEM((_L * 17,), jnp.int32),
            pltpu.VMEM((_L * 17,), jnp.float32),
            pltpu.VMEM((3 * _L,), jnp.float32),
            pltpu.SemaphoreType.DMA,
            pltpu.SemaphoreType.DMA,
            pltpu.SemaphoreType.DMA,
            pltpu.SemaphoreType.DMA,
            pltpu.SemaphoreType.DMA,
            pltpu.SemaphoreType.DMA,
            pltpu.SemaphoreType.DMA,
        ],
    )
    def sc_hist(p_hbm, y_hbm, edges_hbm, out_hbm,
                p_v, y_v, e_v, e_t, cy_t, sp_t, res_v,
                sem_p1, sem_y1, sem_p2, sem_y2, sem_p3, sem_y3, sem_e):
        wid = lax.axis_index("c") * _NS + lax.axis_index("s")
        base = wid * big
        is_big = wid < _NW - 1

        cp_e = pltpu.async_copy(edges_hbm, e_v, sem_e)
        cp_p1 = pltpu.async_copy(
            p_hbm.at[pl.ds(base, half1)], p_v.at[pl.ds(0, half1)], sem_p1)
        cp_y1 = pltpu.async_copy(
            y_hbm.at[pl.ds(base, half1)], y_v.at[pl.ds(0, half1)], sem_y1)
        @pl.when(is_big)
        def _():
            pltpu.async_copy(
                p_hbm.at[pl.ds(base + small, big - small)],
                p_v.at[pl.ds(small, big - small)], sem_p3)
            pltpu.async_copy(
                y_hbm.at[pl.ds(base + small, big - small)],
                y_v.at[pl.ds(small, big - small)], sem_y3)

        zeros_i = jnp.zeros((_L,), jnp.int32)
        zeros_f = jnp.zeros((_L,), jnp.float32)
        for k in range(17):
            cy_t[pl.ds(k * _L, _L)] = zeros_i
            sp_t[pl.ds(k * _L, _L)] = zeros_f

        lane_base = lax.iota(jnp.int32, _L) * 17
        cnt_one = jnp.full((_L,), 65536, jnp.int32)

        def steps_interleaved(base_v, width):
            ss = [pl.ds((base_v + u) * _L, _L) for u in range(width)]
            ps = [p_v[s] for s in ss]
            b0s = [jnp.minimum((p * 15.0).astype(jnp.int32), _N_BINS - 1)
                   for p in ps]
            i0s = [lane_base + b0 for b0 in b0s]
            los = [plsc.load_gather(e_t, [i0]) for i0 in i0s]
            ys = [y_v[s] for s in ss]
            idxs = [jnp.where(ps[u] < los[u], i0s[u] - 1, i0s[u])
                    for u in range(width)]
            for u in range(width):
                cy = cnt_one + ys[u].astype(jnp.int32)
                plsc.addupdate_scatter(cy_t, [idxs[u]], cy)
                plsc.addupdate_scatter(sp_t, [idxs[u]], ps[u])

        def body(i, carry):
            steps_interleaved(i * unroll, unroll)
            return carry

        cp_e.wait()
        ev = e_v[pl.ds(0, _L)]
        for k in range(_L):
            e_t[pl.ds(k * 17, _L)] = ev
        cp_p1.wait()
        cp_y1.wait()
        lax.fori_loop(0, n_common // unroll, body, jnp.int32(0))

        @pl.when(is_big)
        def _():
            pltpu.make_async_copy(
                p_hbm.at[pl.ds(base + small, big - small)],
                p_v.at[pl.ds(small, big - small)], sem_p3).wait()
            pltpu.make_async_copy(
                y_hbm.at[pl.ds(base + small, big - small)],
                y_v.at[pl.ds(small, big - small)], sem_y3).wait()
            for t in range(0, n_extra, 7):
                steps_interleaved(n_common + t, min(7, n_extra - t))

        acc_cy = cy_t[pl.ds(0, _L)]
        acc_sp = sp_t[pl.ds(0, _L)]
        for k in range(1, _L):
            acc_cy = acc_cy + cy_t[pl.ds(k * 17, _L)]
            acc_sp = acc_sp + sp_t[pl.ds(k * 17, _L)]
        res_v[pl.ds(0, _L)] = (
            lax.shift_right_logical(acc_cy, 16).astype(jnp.float32))
        res_v[pl.ds(_L, _L)] = acc_sp
        res_v[pl.ds(2 * _L, _L)] = (acc_cy & 0xFFFF).astype(jnp.float32)
        pltpu.sync_copy(res_v, out_hbm.at[wid])

    return sc_hist


@functools.cache
def _build_epilogue(n):
    def body(x_ref, o_ref):
        s = jnp.sum(x_ref[...], axis=0, keepdims=True)
        cnt = s[:, 0:_L]
        sp = s[:, _L:2 * _L]
        sy = s[:, 2 * _L:3 * _L]
        valid = lax.broadcasted_iota(jnp.int32, (1, _L), 1) < _N_BINS
        cnt = jnp.where(valid, cnt, 0.0)
        sp = jnp.where(valid, sp, 0.0)
        sy = jnp.where(valid, sy, 0.0)
        frac = jnp.sum(sy) / np.float32(n)
        e1 = 2.0 * frac
        e2 = 0.5 + frac
        safe = jnp.maximum(cnt, 1.0)
        p_mean = sp / safe
        y_mean = sy / safe
        lg = jnp.log(safe)
        nonempty = valid & (cnt > 0.0)
        w = jnp.where(nonempty, 0.5 * (jnp.exp(e1 * lg) + jnp.exp(e2 * lg)),
                      0.0)
        num = jnp.abs(p_mean - y_mean) * w
        o_ref[...] = (jnp.sum(num, axis=1, keepdims=True)
                      / jnp.sum(w, axis=1, keepdims=True))

    return pl.pallas_call(
        body, out_shape=jax.ShapeDtypeStruct((1, 1), jnp.float32))


def kernel(pred_probas, y_true):
    n = pred_probas.shape[0]
    edges = np.linspace(0.0, 1.0, _N_BINS + 1).astype(np.float32)
    assert all(np.float32(edges[b] * np.float32(_N_BINS)) == np.float32(b)
               for b in range(_N_BINS + 1))
    partials = _build_sc_hist(n)(pred_probas, y_true, jnp.asarray(edges))
    out = _build_epilogue(n)(partials)
    return out[0, 0]


# speedup vs baseline: 1.0129x; 1.0129x over previous
import functools

import numpy as np
import jax
import jax.numpy as jnp
from jax import lax
from jax.experimental import pallas as pl
from jax.experimental.pallas import tpu as pltpu
from jax.experimental.pallas import tpu_sc as plsc

_N_BINS = 15
_NC = 2
_NS = 16
_L = 16
_NW = _NC * _NS


@functools.cache
def _build_sc_hist(n):
    assert n % _L == 0
    big = ((n + _NW * _L - 1) // (_NW * _L)) * _L
    small = n - (_NW - 1) * big
    assert 0 < small <= big and small % _L == 0
    assert (big // _L) * 2049 < 2**24
    n_common = small // _L
    n_extra = (big - small) // _L
    unroll = 18
    assert n_common % unroll == 0
    half1 = small
    mesh = plsc.VectorSubcoreMesh(core_axis_name="c", subcore_axis_name="s")

    @functools.partial(
        pl.kernel,
        mesh=mesh,
        compiler_params=pltpu.CompilerParams(needs_layout_passes=False),
        out_type=jax.ShapeDtypeStruct((_NW, 3 * _L), jnp.float32),
        scratch_types=[
            pltpu.VMEM((big,), jnp.float32),
            pltpu.VMEM((big,), jnp.float32),
            pltpu.VMEM((_L,), jnp.float32),
            pltpu.VMEM((_L * _L,), jnp.float32),
            pltpu.VMEM((_L * _L,), jnp.float32),
            pltpu.VMEM((3 * _L,), jnp.float32),
            pltpu.SemaphoreType.DMA,
            pltpu.SemaphoreType.DMA,
            pltpu.SemaphoreType.DMA,
            pltpu.SemaphoreType.DMA,
            pltpu.SemaphoreType.DMA,
            pltpu.SemaphoreType.DMA,
            pltpu.SemaphoreType.DMA,
        ],
    )
    def sc_hist(p_hbm, y_hbm, edges_hbm, out_hbm,
                p_v, y_v, e_v, cy_t, sp_t, res_v,
                sem_p1, sem_y1, sem_p2, sem_y2, sem_p3, sem_y3, sem_e):
        wid = lax.axis_index("c") * _NS + lax.axis_index("s")
        base = wid * big
        is_big = wid < _NW - 1

        cp_e = pltpu.async_copy(edges_hbm, e_v, sem_e)
        cp_p1 = pltpu.async_copy(
            p_hbm.at[pl.ds(base, half1)], p_v.at[pl.ds(0, half1)], sem_p1)
        cp_y1 = pltpu.async_copy(
            y_hbm.at[pl.ds(base, half1)], y_v.at[pl.ds(0, half1)], sem_y1)
        @pl.when(is_big)
        def _():
            pltpu.async_copy(
                p_hbm.at[pl.ds(base + small, big - small)],
                p_v.at[pl.ds(small, big - small)], sem_p3)
            pltpu.async_copy(
                y_hbm.at[pl.ds(base + small, big - small)],
                y_v.at[pl.ds(small, big - small)], sem_y3)

        zeros_f = jnp.zeros((_L,), jnp.float32)
        for k in range(_L):
            cy_t[pl.ds(k * _L, _L)] = zeros_f
            sp_t[pl.ds(k * _L, _L)] = zeros_f

        lane_base = lax.iota(jnp.int32, _L) * _L
        cnt_one = jnp.full((_L,), 2048.0, jnp.float32)

        def steps_interleaved(base_v, width):
            ss = [pl.ds((base_v + u) * _L, _L) for u in range(width)]
            ps = [p_v[s] for s in ss]
            b0s = [(p * 15.0).astype(jnp.int32) for p in ps]
            los = [plsc.load_gather(e_v, [b0]) for b0 in b0s]
            ys = [y_v[s] for s in ss]
            idxs = [lane_base + jnp.where(ps[u] < los[u], b0s[u] - 1,
                                          b0s[u])
                    for u in range(width)]
            for u in range(width):
                plsc.addupdate_scatter(cy_t, [idxs[u]], cnt_one + ys[u])
                plsc.addupdate_scatter(sp_t, [idxs[u]], ps[u])

        def body(i, carry):
            steps_interleaved(i * unroll, unroll)
            return carry

        cp_e.wait()
        cp_p1.wait()
        cp_y1.wait()
        lax.fori_loop(0, n_common // unroll, body, jnp.int32(0))

        @pl.when(is_big)
        def _():
            pltpu.make_async_copy(
                p_hbm.at[pl.ds(base + small, big - small)],
                p_v.at[pl.ds(small, big - small)], sem_p3).wait()
            pltpu.make_async_copy(
                y_hbm.at[pl.ds(base + small, big - small)],
                y_v.at[pl.ds(small, big - small)], sem_y3).wait()
            for t in range(0, n_extra, 7):
                steps_interleaved(n_common + t, min(7, n_extra - t))

        acc_cnt = jnp.zeros((_L,), jnp.int32)
        acc_sy = jnp.zeros((_L,), jnp.int32)
        acc_sp = sp_t[pl.ds(0, _L)]
        for k in range(_L):
            ci = cy_t[pl.ds(k * _L, _L)].astype(jnp.int32)
            acc_cnt = acc_cnt + lax.shift_right_logical(ci, 11)
            acc_sy = acc_sy + (ci & 2047)
            if k > 0:
                acc_sp = acc_sp + sp_t[pl.ds(k * _L, _L)]
        res_v[pl.ds(0, _L)] = acc_cnt.astype(jnp.float32)
        res_v[pl.ds(_L, _L)] = acc_sp
        res_v[pl.ds(2 * _L, _L)] = acc_sy.astype(jnp.float32)
        pltpu.sync_copy(res_v, out_hbm.at[wid])

    return sc_hist


@functools.cache
def _build_epilogue(n):
    def body(x_ref, o_ref):
        s = jnp.sum(x_ref[...], axis=0, keepdims=True)
        cnt = s[:, 0:_L]
        sp = s[:, _L:2 * _L]
        sy = s[:, 2 * _L:3 * _L]
        valid = lax.broadcasted_iota(jnp.int32, (1, _L), 1) < _N_BINS
        cnt = jnp.where(valid, cnt, 0.0)
        sp = jnp.where(valid, sp, 0.0)
        sy = jnp.where(valid, sy, 0.0)
        frac = jnp.sum(sy) / np.float32(n)
        e1 = 2.0 * frac
        e2 = 0.5 + frac
        safe = jnp.maximum(cnt, 1.0)
        p_mean = sp / safe
        y_mean = sy / safe
        lg = jnp.log(safe)
        nonempty = valid & (cnt > 0.0)
        w = jnp.where(nonempty, 0.5 * (jnp.exp(e1 * lg) + jnp.exp(e2 * lg)),
                      0.0)
        num = jnp.abs(p_mean - y_mean) * w
        o_ref[...] = (jnp.sum(num, axis=1, keepdims=True)
                      / jnp.sum(w, axis=1, keepdims=True))

    return pl.pallas_call(
        body, out_shape=jax.ShapeDtypeStruct((1, 1), jnp.float32))


def kernel(pred_probas, y_true):
    n = pred_probas.shape[0]
    edges = np.linspace(0.0, 1.0, _N_BINS + 1).astype(np.float32)
    assert all(np.float32(edges[b] * np.float32(_N_BINS)) == np.float32(b)
               for b in range(_N_BINS + 1))
    partials = _build_sc_hist(n)(pred_probas, y_true, jnp.asarray(edges))
    out = _build_epilogue(n)(partials)
    return out[0, 0]


# speedup vs baseline: 1.0546x; 1.0412x over previous
import functools

import numpy as np
import jax
import jax.numpy as jnp
from jax import lax
from jax.experimental import pallas as pl
from jax.experimental.pallas import tpu as pltpu
from jax.experimental.pallas import tpu_sc as plsc

_N_BINS = 15
_NC = 2
_NS = 16
_L = 16
_NW = _NC * _NS


@functools.cache
def _build_sc_hist(n):
    assert n % _L == 0
    big = ((n + _NW * _L - 1) // (_NW * _L)) * _L
    small = n - (_NW - 1) * big
    assert 0 < small <= big and small % _L == 0
    assert big * 65537 < 2**31
    n_common = small // _L
    n_extra = (big - small) // _L
    unroll = 18
    assert n_common % unroll == 0
    half1 = small
    mesh = plsc.VectorSubcoreMesh(core_axis_name="c", subcore_axis_name="s")

    @functools.partial(
        pl.kernel,
        mesh=mesh,
        compiler_params=pltpu.CompilerParams(needs_layout_passes=False),
        out_type=jax.ShapeDtypeStruct((_NW, 3 * _L), jnp.float32),
        scratch_types=[
            pltpu.VMEM((big,), jnp.float32),
            pltpu.VMEM((big,), jnp.float32),
            pltpu.VMEM((_L,), jnp.float32),
            pltpu.VMEM((_L * _L,), jnp.int32),
            pltpu.VMEM((_L * _L,), jnp.float32),
            pltpu.VMEM((3 * _L,), jnp.float32),
            pltpu.SemaphoreType.DMA,
            pltpu.SemaphoreType.DMA,
            pltpu.SemaphoreType.DMA,
            pltpu.SemaphoreType.DMA,
            pltpu.SemaphoreType.DMA,
            pltpu.SemaphoreType.DMA,
            pltpu.SemaphoreType.DMA,
        ],
    )
    def sc_hist(p_hbm, y_hbm, edges_hbm, out_hbm,
                p_v, y_v, e_v, cy_t, sp_t, res_v,
                sem_p1, sem_y1, sem_p2, sem_y2, sem_p3, sem_y3, sem_e):
        wid = lax.axis_index("c") * _NS + lax.axis_index("s")
        base = wid * big
        is_big = wid < _NW - 1

        cp_e = pltpu.async_copy(edges_hbm, e_v, sem_e)
        cp_p1 = pltpu.async_copy(
            p_hbm.at[pl.ds(base, half1)], p_v.at[pl.ds(0, half1)], sem_p1)
        cp_y1 = pltpu.async_copy(
            y_hbm.at[pl.ds(base, half1)], y_v.at[pl.ds(0, half1)], sem_y1)
        @pl.when(is_big)
        def _():
            pltpu.async_copy(
                p_hbm.at[pl.ds(base + small, big - small)],
                p_v.at[pl.ds(small, big - small)], sem_p3)
            pltpu.async_copy(
                y_hbm.at[pl.ds(base + small, big - small)],
                y_v.at[pl.ds(small, big - small)], sem_y3)

        zeros_i = jnp.zeros((_L,), jnp.int32)
        zeros_f = jnp.zeros((_L,), jnp.float32)
        for k in range(_L):
            cy_t[pl.ds(k * _L, _L)] = zeros_i
            sp_t[pl.ds(k * _L, _L)] = zeros_f

        lane_base = lax.iota(jnp.int32, _L) * _L
        cnt_one = jnp.full((_L,), 65536, jnp.int32)

        def steps_interleaved(base_v, width):
            ss = [pl.ds((base_v + u) * _L, _L) for u in range(width)]
            ps = [p_v[s] for s in ss]
            b0s = [(p * 15.0).astype(jnp.int32) for p in ps]
            los = [plsc.load_gather(e_v, [b0]) for b0 in b0s]
            ys = [y_v[s] for s in ss]
            idxs = [lane_base + jnp.where(ps[u] < los[u], b0s[u] - 1,
                                          b0s[u])
                    for u in range(width)]
            for u in range(width):
                cy = cnt_one + ys[u].astype(jnp.int32)
                plsc.addupdate_scatter(cy_t, [idxs[u]], cy)
                plsc.addupdate_scatter(sp_t, [idxs[u]], ps[u])

        def body(i, carry):
            steps_interleaved(i * unroll, unroll)
            return carry

        cp_e.wait()
        cp_p1.wait()
        cp_y1.wait()
        lax.fori_loop(0, n_common // unroll, body, jnp.int32(0))

        @pl.when(is_big)
        def _():
            pltpu.make_async_copy(
                p_hbm.at[pl.ds(base + small, big - small)],
                p_v.at[pl.ds(small, big - small)], sem_p3).wait()
            pltpu.make_async_copy(
                y_hbm.at[pl.ds(base + small, big - small)],
                y_v.at[pl.ds(small, big - small)], sem_y3).wait()
            for t in range(0, n_extra, 7):
                steps_interleaved(n_common + t, min(7, n_extra - t))

        acc_cy = cy_t[pl.ds(0, _L)]
        acc_sp = sp_t[pl.ds(0, _L)]
        for k in range(1, _L):
            acc_cy = acc_cy + cy_t[pl.ds(k * _L, _L)]
            acc_sp = acc_sp + sp_t[pl.ds(k * _L, _L)]
        res_v[pl.ds(0, _L)] = (
            lax.shift_right_logical(acc_cy, 16).astype(jnp.float32))
        res_v[pl.ds(_L, _L)] = acc_sp
        res_v[pl.ds(2 * _L, _L)] = (acc_cy & 0xFFFF).astype(jnp.float32)
        pltpu.sync_copy(res_v, out_hbm.at[wid])

    return sc_hist


@functools.cache
def _build_epilogue(n):
    def body(x_ref, o_ref):
        s = jnp.sum(x_ref[...], axis=0, keepdims=True)
        cnt = s[:, 0:_L]
        sp = s[:, _L:2 * _L]
        sy = s[:, 2 * _L:3 * _L]
        valid = lax.broadcasted_iota(jnp.int32, (1, _L), 1) < _N_BINS
        cnt = jnp.where(valid, cnt, 0.0)
        sp = jnp.where(valid, sp, 0.0)
        sy = jnp.where(valid, sy, 0.0)
        frac = jnp.sum(sy) / np.float32(n)
        e1 = 2.0 * frac
        e2 = 0.5 + frac
        safe = jnp.maximum(cnt, 1.0)
        p_mean = sp / safe
        y_mean = sy / safe
        lg = jnp.log(safe)
        nonempty = valid & (cnt > 0.0)
        w = jnp.where(nonempty, 0.5 * (jnp.exp(e1 * lg) + jnp.exp(e2 * lg)),
                      0.0)
        num = jnp.abs(p_mean - y_mean) * w
        o_ref[...] = (jnp.sum(num, axis=1, keepdims=True)
                      / jnp.sum(w, axis=1, keepdims=True))

    return pl.pallas_call(
        body, out_shape=jax.ShapeDtypeStruct((1, 1), jnp.float32))


def kernel(pred_probas, y_true):
    n = pred_probas.shape[0]
    edges = np.linspace(0.0, 1.0, _N_BINS + 1).astype(np.float32)
    assert all(np.float32(edges[b] * np.float32(_N_BINS)) == np.float32(b)
               for b in range(_N_BINS + 1))
    partials = _build_sc_hist(n)(pred_probas, y_true, jnp.asarray(edges))
    out = _build_epilogue(n)(partials)
    return out[0, 0]


# speedup vs baseline: 1.0685x; 1.0131x over previous
import functools

import numpy as np
import jax
import jax.numpy as jnp
from jax import lax
from jax.experimental import pallas as pl
from jax.experimental.pallas import tpu as pltpu
from jax.experimental.pallas import tpu_sc as plsc

_N_BINS = 15
_NC = 2
_NS = 16
_L = 16
_NW = _NC * _NS


@functools.cache
def _build_sc_hist(n):
    assert n % _L == 0
    big = ((n + _NW * _L - 1) // (_NW * _L)) * _L
    small = n - (_NW - 1) * big
    assert 0 < small <= big and small % _L == 0
    assert big * 65537 < 2**31
    n_common = small // _L
    n_extra = (big - small) // _L
    unroll = 18
    assert n_common % unroll == 0
    half1 = small
    mesh = plsc.VectorSubcoreMesh(core_axis_name="c", subcore_axis_name="s")

    @functools.partial(
        pl.kernel,
        mesh=mesh,
        compiler_params=pltpu.CompilerParams(needs_layout_passes=False),
        out_type=jax.ShapeDtypeStruct((_NW, 3 * _L), jnp.float32),
        scratch_types=[
            pltpu.VMEM((big,), jnp.float32),
            pltpu.VMEM((big,), jnp.float32),
            pltpu.VMEM((_L,), jnp.float32),
            pltpu.VMEM((_L * _L,), jnp.int32),
            pltpu.VMEM((_L * _L,), jnp.float32),
            pltpu.VMEM((3 * _L,), jnp.float32),
            pltpu.SemaphoreType.DMA,
            pltpu.SemaphoreType.DMA,
            pltpu.SemaphoreType.DMA,
            pltpu.SemaphoreType.DMA,
            pltpu.SemaphoreType.DMA,
            pltpu.SemaphoreType.DMA,
            pltpu.SemaphoreType.DMA,
        ],
    )
    def sc_hist(p_hbm, y_hbm, edges_hbm, out_hbm,
                p_v, y_v, e_v, cy_t, sp_t, res_v,
                sem_p1, sem_y1, sem_p2, sem_y2, sem_p3, sem_y3, sem_e):
        wid = lax.axis_index("c") * _NS + lax.axis_index("s")
        base = wid * big
        is_big = wid < _NW - 1

        cp_e = pltpu.async_copy(edges_hbm, e_v, sem_e)
        cp_p1 = pltpu.async_copy(
            p_hbm.at[pl.ds(base, half1)], p_v.at[pl.ds(0, half1)], sem_p1)
        cp_y1 = pltpu.async_copy(
            y_hbm.at[pl.ds(base, half1)], y_v.at[pl.ds(0, half1)], sem_y1)
        @pl.when(is_big)
        def _():
            pltpu.async_copy(
                p_hbm.at[pl.ds(base + small, big - small)],
                p_v.at[pl.ds(small, big - small)], sem_p3)
            pltpu.async_copy(
                y_hbm.at[pl.ds(base + small, big - small)],
                y_v.at[pl.ds(small, big - small)], sem_y3)

        zeros_i = jnp.zeros((_L,), jnp.int32)
        zeros_f = jnp.zeros((_L,), jnp.float32)
        for k in range(_L):
            cy_t[pl.ds(k * _L, _L)] = zeros_i
            sp_t[pl.ds(k * _L, _L)] = zeros_f

        lane_base = lax.iota(jnp.int32, _L) * _L
        cnt_one = jnp.full((_L,), 65536, jnp.int32)

        def steps_interleaved(base_v, width):
            ss = [pl.ds((base_v + u) * _L, _L) for u in range(width)]
            ps = [p_v[s] for s in ss]
            b0s = [jnp.minimum((p * 15.0).astype(jnp.int32), _N_BINS - 1)
                   for p in ps]
            los = [plsc.load_gather(e_v, [b0]) for b0 in b0s]
            ys = [y_v[s] for s in ss]
            idxs = [lane_base + jnp.where(ps[u] < los[u], b0s[u] - 1,
                                          b0s[u])
                    for u in range(width)]
            for u in range(width):
                cy = cnt_one + ys[u].astype(jnp.int32)
                plsc.addupdate_scatter(cy_t, [idxs[u]], cy)
                plsc.addupdate_scatter(sp_t, [idxs[u]], ps[u])

        def body(i, carry):
            steps_interleaved(i * unroll, unroll)
            return carry

        cp_e.wait()
        cp_p1.wait()
        cp_y1.wait()
        lax.fori_loop(0, n_common // unroll, body, jnp.int32(0))

        @pl.when(is_big)
        def _():
            pltpu.make_async_copy(
                p_hbm.at[pl.ds(base + small, big - small)],
                p_v.at[pl.ds(small, big - small)], sem_p3).wait()
            pltpu.make_async_copy(
                y_hbm.at[pl.ds(base + small, big - small)],
                y_v.at[pl.ds(small, big - small)], sem_y3).wait()
            for t in range(0, n_extra, 7):
                steps_interleaved(n_common + t, min(7, n_extra - t))

        acc_cy = cy_t[pl.ds(0, _L)]
        acc_sp = sp_t[pl.ds(0, _L)]
        for k in range(1, _L):
            acc_cy = acc_cy + cy_t[pl.ds(k * _L, _L)]
            acc_sp = acc_sp + sp_t[pl.ds(k * _L, _L)]
        res_v[pl.ds(0, _L)] = (
            lax.shift_right_logical(acc_cy, 16).astype(jnp.float32))
        res_v[pl.ds(_L, _L)] = acc_sp
        res_v[pl.ds(2 * _L, _L)] = (acc_cy & 0xFFFF).astype(jnp.float32)
        pltpu.sync_copy(res_v, out_hbm.at[wid])

    return sc_hist


@functools.cache
def _build_epilogue(n):
    def body(x_ref, o_ref):
        s = jnp.sum(x_ref[...], axis=0, keepdims=True)
        cnt = s[:, 0:_L]
        sp = s[:, _L:2 * _L]
        sy = s[:, 2 * _L:3 * _L]
        valid = lax.broadcasted_iota(jnp.int32, (1, _L), 1) < _N_BINS
        cnt = jnp.where(valid, cnt, 0.0)
        sp = jnp.where(valid, sp, 0.0)
        sy = jnp.where(valid, sy, 0.0)
        frac = jnp.sum(sy) / np.float32(n)
        e1 = 2.0 * frac
        e2 = 0.5 + frac
        safe = jnp.maximum(cnt, 1.0)
        p_mean = sp / safe
        y_mean = sy / safe
        lg = jnp.log(safe)
        nonempty = valid & (cnt > 0.0)
        w = jnp.where(nonempty, 0.5 * (jnp.exp(e1 * lg) + jnp.exp(e2 * lg)),
                      0.0)
        num = jnp.abs(p_mean - y_mean) * w
        o_ref[...] = (jnp.sum(num, axis=1, keepdims=True)
                      / jnp.sum(w, axis=1, keepdims=True))

    return pl.pallas_call(
        body, out_shape=jax.ShapeDtypeStruct((1, 1), jnp.float32))


def kernel(pred_probas, y_true):
    n = pred_probas.shape[0]
    edges = np.linspace(0.0, 1.0, _N_BINS + 1).astype(np.float32)
    assert all(np.float32(edges[b] * np.float32(_N_BINS)) == np.float32(b)
               for b in range(_N_BINS + 1))
    partials = _build_sc_hist(n)(pred_probas, y_true, jnp.asarray(edges))
    out = _build_epilogue(n)(partials)
    return out[0, 0]
